# Initial kernel scaffold; baseline (speedup 1.0000x reference)
import jax
import jax.numpy as jnp
from jax.experimental import pallas as pl


def kernel(x, edge_index, edge_attr, W1, b1, W2, b2):
    raise NotImplementedError("write your pallas kernel here")


# speedup vs baseline: 2.3557x; 2.3557x over previous
import functools

import jax
import jax.numpy as jnp
from jax import lax
from jax.experimental import pallas as pl
from jax.experimental.pallas import tpu as pltpu
from jax.experimental.pallas import tpu_sc as plsc

N_NODES = 10000
N_EDGES = 320000
DIM_NODE = 128
DIM_EDGE = 16
HID = 2 * DIM_NODE
NEG_SLOPE = 0.2

_NC = 2
_NS = 16
_NW = _NC * _NS

_B = 2 * N_EDGES
_BPW = _B // _NW
_CHUNK = 80
_NCHUNK = _BPW // _CHUNK


def _sc_gather(table, idx3):
    mesh = plsc.VectorSubcoreMesh(core_axis_name="c", subcore_axis_name="s")

    @functools.partial(
        pl.kernel,
        mesh=mesh,
        out_type=jax.ShapeDtypeStruct((_B, DIM_NODE), jnp.float32),
        scratch_types=[
            pltpu.VMEM((_NCHUNK, _CHUNK), jnp.int32),
            pltpu.VMEM((_CHUNK, DIM_NODE), jnp.float32),
            pltpu.SemaphoreType.DMA,
        ],
    )
    def gather_kernel(table_hbm, idx_hbm, out_hbm, idx_v, rows_v, sem):
        wid = lax.axis_index("s") * _NC + lax.axis_index("c")
        base = wid * _BPW
        pltpu.sync_copy(idx_hbm.at[wid], idx_v)

        def body(i, carry):
            pltpu.async_copy(table_hbm.at[idx_v.at[i]], rows_v, sem).wait()
            pltpu.sync_copy(rows_v, out_hbm.at[pl.ds(base + i * _CHUNK, _CHUNK)])
            return carry

        lax.fori_loop(0, _NCHUNK, body, 0)

    return gather_kernel(table, idx3)


_E_BLK = 2560
_NB = N_EDGES // _E_BLK


def _mlp_body(xi_ref, xj_ref, ea_ref, wa_ref, wb_ref, wc_ref, b1_ref, w2_ref,
              b2_ref, out_ref):
    h = jnp.dot(xi_ref[...], wa_ref[...], preferred_element_type=jnp.float32)
    h += jnp.dot(xj_ref[...], wb_ref[...], preferred_element_type=jnp.float32)
    h += jnp.dot(ea_ref[...], wc_ref[...], preferred_element_type=jnp.float32)
    h += b1_ref[...]
    h = jnp.where(h >= 0, h, NEG_SLOPE * h)
    out_ref[...] = (
        jnp.sum(h * w2_ref[...], axis=1, keepdims=True) + b2_ref[...]
    )


def _tc_mlp(g, ea, wa, wb, wc, b1, w2, b2):
    grid = (_NB,)
    return pl.pallas_call(
        _mlp_body,
        grid=grid,
        in_specs=[
            pl.BlockSpec((_E_BLK, DIM_NODE), lambda i: (i, 0)),
            pl.BlockSpec((_E_BLK, DIM_NODE), lambda i: (i + _NB, 0)),
            pl.BlockSpec((_E_BLK, DIM_EDGE), lambda i: (i, 0)),
            pl.BlockSpec((DIM_NODE, HID), lambda i: (0, 0)),
            pl.BlockSpec((DIM_NODE, HID), lambda i: (0, 0)),
            pl.BlockSpec((DIM_EDGE, HID), lambda i: (0, 0)),
            pl.BlockSpec((1, HID), lambda i: (0, 0)),
            pl.BlockSpec((1, HID), lambda i: (0, 0)),
            pl.BlockSpec((1, 1), lambda i: (0, 0)),
        ],
        out_specs=pl.BlockSpec((_E_BLK, 1), lambda i: (i, 0)),
        out_shape=jax.ShapeDtypeStruct((N_EDGES, 1), jnp.float32),
    )(g, g, ea, wa, wb, wc, b1, w2, b2)


def kernel(x, edge_index, edge_attr, W1, b1, W2, b2):
    src = edge_index[0, :].astype(jnp.int32)
    dst = edge_index[1, :].astype(jnp.int32)
    idx3 = jnp.concatenate([src, dst]).reshape(_NW, _NCHUNK, _CHUNK)

    g = _sc_gather(x, idx3)

    w1t = W1.T
    wa = w1t[:DIM_NODE]
    wb = w1t[DIM_NODE:2 * DIM_NODE]
    wc = w1t[2 * DIM_NODE:]
    return _tc_mlp(
        g,
        edge_attr,
        wa,
        wb,
        wc,
        b1.reshape(1, HID),
        W2.reshape(1, HID),
        b2.reshape(1, 1),
    )
